# baseline (speedup 1.0000x reference)
import functools

import jax
import jax.numpy as jnp
import numpy as np
from jax import lax
from jax.experimental import pallas as pl
from jax.experimental.pallas import tpu as pltpu
from jax.experimental.pallas import tpu_sc as plsc

N_TOK = 32768
N_EMBED = 1024
N_EXPERTS = 8
TOP_K = 2

BLOCK_T = 2048

_NEG_INF = float("-inf")

_NC = 2
_NS = 16
_NW = _NC * _NS
_TPW = N_TOK // _NW
_L = 16
_GPW = _TPW // _L


def _mixed_block(x_ref, w_ref, b_ref, noise_ref, mixed_ref):
    acc = jnp.dot(x_ref[...], w_ref[...], preferred_element_type=jnp.float32)
    acc_t = acc.T + b_ref[...]
    logits = acc_t[:N_EXPERTS, :]
    pre = acc_t[N_EXPERTS:, :]
    mixed_ref[...] = logits + noise_ref[...] * jax.nn.softplus(pre)


def _routing_sc(mixed_hbm, sparse_hbm, idx_hbm, full_hbm, mvm, svm, ivm, fvm):
    wid = lax.axis_index("s") * _NC + lax.axis_index("c")
    tbase = wid * _TPW
    for e in range(N_EXPERTS):
        pltpu.sync_copy(mixed_hbm.at[e, pl.ds(tbase, _TPW)], mvm.at[e])

    def group(g, carry):
        sl = pl.ds(g * _L, _L)
        v = [mvm[e, sl] for e in range(N_EXPERTS)]

        v1 = v[0]
        i1 = jnp.zeros((_L,), jnp.int32)
        for e in range(1, N_EXPERTS):
            gt = v[e] > v1
            v1 = jnp.where(gt, v[e], v1)
            i1 = jnp.where(gt, e, i1)
        v2 = jnp.where(i1 == 0, _NEG_INF, v[0])
        i2 = jnp.zeros((_L,), jnp.int32)
        for e in range(1, N_EXPERTS):
            ve = jnp.where(i1 == e, _NEG_INF, v[e])
            gt = ve > v2
            v2 = jnp.where(gt, ve, v2)
            i2 = jnp.where(gt, e, i2)
        ivm[0, sl] = i1
        ivm[1, sl] = i2

        ex = [jnp.exp(v[e] - v1) for e in range(N_EXPERTS)]
        s = ex[0]
        for e in range(1, N_EXPERTS):
            s = s + ex[e]
        r = 1.0 / s
        e2 = jnp.exp(v2 - v1)
        p1 = 1.0 / (1.0 + e2)
        p2 = e2 * p1
        for e in range(N_EXPERTS):
            fvm[e, sl] = ex[e] * r
            svm[e, sl] = jnp.where(
                i1 == e, p1, jnp.where(i2 == e, p2, 0.0)
            )
        return carry

    lax.fori_loop(0, _GPW, group, 0)

    for e in range(N_EXPERTS):
        pltpu.sync_copy(svm.at[e], sparse_hbm.at[e, pl.ds(tbase, _TPW)])
        pltpu.sync_copy(fvm.at[e], full_hbm.at[e, pl.ds(tbase, _TPW)])
    pltpu.sync_copy(ivm.at[0], idx_hbm.at[0, pl.ds(tbase, _TPW)])
    pltpu.sync_copy(ivm.at[1], idx_hbm.at[1, pl.ds(tbase, _TPW)])


_NOISE_CACHE = []


def _fixed_noise():
    if not _NOISE_CACHE:
        with jax.ensure_compile_time_eval():
            raw = jax.random.normal(jax.random.key(42), (N_TOK, N_EXPERTS), jnp.float32)
        _NOISE_CACHE.append(np.asarray(raw).T.copy())
    return _NOISE_CACHE[0]


@functools.partial(jax.jit, static_argnums=())
def kernel(x, W1, b1, W2, b2):
    w = jnp.concatenate([W1, W2], axis=1)
    b = jnp.concatenate([b1, b2])[:, None]
    noise_t = jnp.asarray(_fixed_noise())

    grid = (N_TOK // BLOCK_T,)
    mixed_t = pl.pallas_call(
        _mixed_block,
        grid=grid,
        in_specs=[
            pl.BlockSpec((BLOCK_T, N_EMBED), lambda i: (i, 0)),
            pl.BlockSpec((N_EMBED, 2 * N_EXPERTS), lambda i: (0, 0)),
            pl.BlockSpec((2 * N_EXPERTS, 1), lambda i: (0, 0)),
            pl.BlockSpec((N_EXPERTS, BLOCK_T), lambda i: (0, i)),
        ],
        out_specs=pl.BlockSpec((N_EXPERTS, BLOCK_T), lambda i: (0, i)),
        out_shape=jax.ShapeDtypeStruct((N_EXPERTS, N_TOK), jnp.float32),
    )(x, w, b, noise_t)

    sc = functools.partial(
        pl.kernel,
        mesh=plsc.VectorSubcoreMesh(core_axis_name="c", subcore_axis_name="s"),
        out_type=[
            jax.ShapeDtypeStruct((N_EXPERTS, N_TOK), jnp.float32),
            jax.ShapeDtypeStruct((TOP_K, N_TOK), jnp.int32),
            jax.ShapeDtypeStruct((N_EXPERTS, N_TOK), jnp.float32),
        ],
        scratch_types=[
            pltpu.VMEM((N_EXPERTS, _TPW), jnp.float32),
            pltpu.VMEM((N_EXPERTS, _TPW), jnp.float32),
            pltpu.VMEM((TOP_K, _TPW), jnp.int32),
            pltpu.VMEM((N_EXPERTS, _TPW), jnp.float32),
        ],
    )(_routing_sc)
    sparse_t, idx_t, full_t = sc(mixed_t)

    return (sparse_t.T, idx_t.T, full_t.T)


# speedup vs baseline: 1.5350x; 1.5350x over previous
import functools

import jax
import jax.numpy as jnp
import numpy as np
from jax.experimental import pallas as pl

N_TOK = 32768
N_EMBED = 1024
N_EXPERTS = 8
TOP_K = 2

BLOCK_T = 2048

_NEG_INF = float("-inf")


def _router_block(x_ref, w_ref, b_ref, noise_ref, sparse_ref, idx_ref, full_ref):
    acc = jnp.dot(x_ref[...], w_ref[...], preferred_element_type=jnp.float32)
    acc_t = acc.T + b_ref[...]
    logits = acc_t[:N_EXPERTS, :]
    pre = acc_t[N_EXPERTS:, :]
    mixed = logits + noise_ref[...] * jax.nn.softplus(pre)

    m = jnp.max(mixed, axis=0, keepdims=True)
    e = jnp.exp(mixed - m)
    full_ref[...] = e * (1.0 / jnp.sum(e, axis=0, keepdims=True))

    rows = jax.lax.broadcasted_iota(jnp.int32, mixed.shape, 0)
    i1 = jnp.min(jnp.where(mixed == m, rows, N_EXPERTS), axis=0, keepdims=True)
    masked = jnp.where(rows == i1, _NEG_INF, mixed)
    v2 = jnp.max(masked, axis=0, keepdims=True)
    i2 = jnp.min(jnp.where(masked == v2, rows, N_EXPERTS), axis=0, keepdims=True)
    idx_ref[...] = jnp.concatenate([i1, i2], axis=0)

    e2 = jnp.exp(v2 - m)
    p = 1.0 / (1.0 + e2)
    sparse_ref[...] = jnp.where(
        rows == i1, p, jnp.where(rows == i2, e2 * p, 0.0)
    )


_NOISE_CACHE = []


def _fixed_noise():
    if not _NOISE_CACHE:
        with jax.ensure_compile_time_eval():
            raw = jax.random.normal(jax.random.key(42), (N_TOK, N_EXPERTS), jnp.float32)
        _NOISE_CACHE.append(np.asarray(raw).T.copy())
    return _NOISE_CACHE[0]


@functools.partial(jax.jit, static_argnums=())
def kernel(x, W1, b1, W2, b2):
    w = jnp.concatenate([W1, W2], axis=1)
    b = jnp.concatenate([b1, b2])[:, None]
    noise_t = jnp.asarray(_fixed_noise())

    grid = (N_TOK // BLOCK_T,)
    sparse_t, idx_t, full_t = pl.pallas_call(
        _router_block,
        grid=grid,
        in_specs=[
            pl.BlockSpec((BLOCK_T, N_EMBED), lambda i: (i, 0)),
            pl.BlockSpec((N_EMBED, 2 * N_EXPERTS), lambda i: (0, 0)),
            pl.BlockSpec((2 * N_EXPERTS, 1), lambda i: (0, 0)),
            pl.BlockSpec((N_EXPERTS, BLOCK_T), lambda i: (0, i)),
        ],
        out_specs=[
            pl.BlockSpec((N_EXPERTS, BLOCK_T), lambda i: (0, i)),
            pl.BlockSpec((TOP_K, BLOCK_T), lambda i: (0, i)),
            pl.BlockSpec((N_EXPERTS, BLOCK_T), lambda i: (0, i)),
        ],
        out_shape=[
            jax.ShapeDtypeStruct((N_EXPERTS, N_TOK), jnp.float32),
            jax.ShapeDtypeStruct((TOP_K, N_TOK), jnp.int32),
            jax.ShapeDtypeStruct((N_EXPERTS, N_TOK), jnp.float32),
        ],
    )(x, w, b, noise_t)
    return (sparse_t.T, idx_t.T, full_t.T)
